# baseline (speedup 1.0000x reference)
import functools

import jax
import jax.numpy as jnp
from jax import lax
from jax.experimental import pallas as pl
from jax.experimental.pallas import tpu as pltpu
from jax.experimental.pallas import tpu_sc as plsc

_L = 16


@functools.lru_cache(maxsize=None)
def _build(B, N, M, E):
    info = plsc.get_sparse_core_info()
    NC, NS = info.num_cores, info.num_subcores
    BPC = B // NC
    EPT = E // NS
    K = 800
    NCH = EPT // K
    assert EPT % K == 0 and K % _L == 0
    NGRP = (NCH + 3) // 4
    G = K // _L
    UNROLL = 10
    assert G % UNROLL == 0
    assert N % 2 == 0

    STRIPE = 6256
    LAST = M - (NS - 1) * STRIPE

    mesh = plsc.VectorSubcoreMesh(core_axis_name="c", subcore_axis_name="s")

    @functools.partial(
        pl.kernel,
        mesh=mesh,
        out_type=jax.ShapeDtypeStruct((B * M,), jnp.float32),
        compiler_params=pltpu.CompilerParams(needs_layout_passes=False),
        scratch_types=(
            [pltpu.VMEM((N // 2,), jnp.int32)]
            + [pltpu.VMEM((K,), jnp.int32) for _ in range(2)]
            + [pltpu.VMEM((K,), jnp.float32) for _ in range(2)]
            + [pltpu.VMEM((K,), jnp.int32) for _ in range(4)]
            + [pltpu.VMEM((K,), jnp.float32) for _ in range(4)]
            + [pltpu.VMEM_SHARED((M,), jnp.float32) for _ in range(BPC)]
            + [pltpu.SemaphoreType.DMA for _ in range(8)]
        ),
    )
    def k(x_hbm, vals_hbm, src_hbm, dst_hbm, bias_hbm, out_hbm,
          x_v, src0, src1, vals0, vals1,
          dst0, dst1, dst2, dst3, msgs0, msgs1, msgs2, msgs3,
          *accs_and_sems):
        accs = accs_and_sems[:BPC]
        sem_in = accs_and_sems[BPC:BPC + 4]
        sem_sc = accs_and_sems[BPC + 4:BPC + 8]
        srcs = (src0, src1)
        valss = (vals0, vals1)
        dsts = (dst0, dst1, dst2, dst3)
        msgss = (msgs0, msgs1, msgs2, msgs3)

        c = lax.axis_index("c")
        s = lax.axis_index("s")
        off = s * STRIPE
        ebase = s * EPT

        def striped_parts(stripe_len):
            parts, p = [], 0
            while p < stripe_len:
                plen = min(K, stripe_len - p)
                parts.append((p, plen))
                p += plen
            return parts

        def seed_stripe(acc, stripe_len):
            for p, plen in striped_parts(stripe_len):
                pltpu.sync_copy(bias_hbm.at[pl.ds(off + p, plen)],
                                msgs0.at[pl.ds(0, plen)])
                pltpu.sync_copy(msgs0.at[pl.ds(0, plen)],
                                acc.at[pl.ds(off + p, plen)])

        for acc in accs:
            @pl.when(s < NS - 1)
            def _():
                seed_stripe(acc, STRIPE)

            @pl.when(s == NS - 1)
            def _():
                seed_stripe(acc, LAST)

        plsc.subcore_barrier()

        def in_descs(i, s2, s4):
            base = ebase + i * K
            sem = sem_in[s4]
            return (
                pltpu.make_async_copy(src_hbm.at[pl.ds(base, K)], srcs[s2], sem),
                pltpu.make_async_copy(vals_hbm.at[pl.ds(base, K)], valss[s2], sem),
                pltpu.make_async_copy(dst_hbm.at[pl.ds(base, K)], dsts[s4], sem),
            )

        def phase(acc, bg):
            for d in in_descs(0, 0, 0):
                d.start()
            for d in in_descs(1, 1, 1):
                d.start()
            pltpu.sync_copy(x_hbm.at[pl.ds(bg * (N // 2), N // 2)], x_v)

            def group_body(g, carry):
                for u in range(4):
                    i = 4 * g + u

                    @pl.when(i < NCH)
                    def _():
                        s2, s4 = u % 2, u
                        for d in in_descs(i, s2, s4):
                            d.wait()

                        src_v, vals_v, msgs_v = srcs[s2], valss[s2], msgss[s4]

                        @plsc.parallel_loop(0, G, 1, unroll=UNROLL)
                        def _(g2):
                            sl = pl.ds(g2 * _L, _L)
                            sv = src_v[sl]
                            packed = plsc.load_gather(
                                x_v, [lax.shift_right_logical(sv, 1)])
                            odd = lax.bitwise_and(sv, 1)
                            xbits = lax.select(
                                odd == 1,
                                lax.bitwise_and(packed, jnp.int32(-65536)),
                                lax.shift_left(packed, 16))
                            xv = plsc.bitcast(xbits, jnp.float32)
                            msgs_v[sl] = xv * vals_v[sl]

                        pltpu.async_copy(msgs_v, acc.at[dsts[s4]], sem_sc[s4],
                                         add=True)

                        n4 = (u + 2) % 4
                        @pl.when(i >= 2)
                        def _():
                            pltpu.make_async_copy(
                                msgss[n4], acc.at[dsts[n4]], sem_sc[n4]).wait()

                        @pl.when(i + 2 < NCH)
                        def _():
                            for d in in_descs(i + 2, (u + 2) % 2, n4):
                                d.start()
                return carry

            lax.fori_loop(0, NGRP, group_body, 0)

            for s4 in ((NCH - 2) % 4, (NCH - 1) % 4):
                pltpu.make_async_copy(
                    msgss[s4], acc.at[dsts[s4]], sem_sc[s4]).wait()

        for b_local in range(BPC):
            phase(accs[b_local], c * BPC + b_local)

        plsc.subcore_barrier()

        def drain_stripe(acc, bg, stripe_len):
            for p, plen in striped_parts(stripe_len):
                pltpu.sync_copy(acc.at[pl.ds(off + p, plen)],
                                msgs0.at[pl.ds(0, plen)])
                pltpu.sync_copy(msgs0.at[pl.ds(0, plen)],
                                out_hbm.at[pl.ds(bg * M + off + p, plen)])

        for b_local in range(BPC):
            acc = accs[b_local]
            bg = c * BPC + b_local

            @pl.when(s < NS - 1)
            def _():
                drain_stripe(acc, bg, STRIPE)

            @pl.when(s == NS - 1)
            def _():
                drain_stripe(acc, bg, LAST)

    return k


def kernel(x, values, bias, indices):
    B, N, _ = x.shape
    M = bias.shape[0]
    E = values.shape[0]
    xb = x[:, :, 0].astype(jnp.bfloat16)
    xp = jax.lax.bitcast_convert_type(
        xb.reshape(B, N // 2, 2), jnp.int32).reshape(B * (N // 2))
    src = indices[0].astype(jnp.int32)
    dst = indices[1].astype(jnp.int32)
    out = _build(B, N, M, E)(xp, values, src, dst, bias[:, 0])
    return out.reshape(B, M, 1)


# speedup vs baseline: 2.9412x; 2.9412x over previous
import functools

import jax
import jax.numpy as jnp
from jax import lax
from jax.experimental import pallas as pl
from jax.experimental.pallas import tpu as pltpu
from jax.experimental.pallas import tpu_sc as plsc

_L = 16


@functools.lru_cache(maxsize=None)
def _build(B, N, M, E):
    info = plsc.get_sparse_core_info()
    NC, NS = info.num_cores, info.num_subcores
    BPC = B // NC
    assert BPC == 2, "x packing assumes two batch rows per SparseCore"
    EPT = E // NS
    K = 800
    NCH = EPT // K
    assert EPT % K == 0 and K % _L == 0
    NGRP = (NCH + 3) // 4
    G = K // _L
    UNROLL = 10
    assert G % UNROLL == 0

    STRIPE = 6256
    LAST = M - (NS - 1) * STRIPE

    mesh = plsc.VectorSubcoreMesh(core_axis_name="c", subcore_axis_name="s")

    @functools.partial(
        pl.kernel,
        mesh=mesh,
        out_type=jax.ShapeDtypeStruct((B * M,), jnp.float32),
        compiler_params=pltpu.CompilerParams(needs_layout_passes=False),
        scratch_types=(
            [pltpu.VMEM((N,), jnp.int32)]
            + [pltpu.VMEM((K,), jnp.int32) for _ in range(2)]
            + [pltpu.VMEM((K,), jnp.float32) for _ in range(2)]
            + [pltpu.VMEM((K,), jnp.int32) for _ in range(4)]
            + [pltpu.VMEM((K,), jnp.float32) for _ in range(4)]
            + [pltpu.VMEM((K,), jnp.float32) for _ in range(4)]
            + [pltpu.VMEM_SHARED((M,), jnp.float32) for _ in range(BPC)]
            + [pltpu.SemaphoreType.DMA for _ in range(8)]
        ),
    )
    def k(x_hbm, vals_hbm, src_hbm, dst_hbm, bias_hbm, out_hbm,
          x_v, src0, src1, vals0, vals1, dst0, dst1, dst2, dst3,
          ma0, ma1, ma2, ma3, mb0, mb1, mb2, mb3,
          *accs_and_sems):
        accs = accs_and_sems[:BPC]
        sem_in = accs_and_sems[BPC:BPC + 4]
        sem_sc = accs_and_sems[BPC + 4:BPC + 8]
        srcs = (src0, src1)
        valss = (vals0, vals1)
        dsts = (dst0, dst1, dst2, dst3)
        msgsa = (ma0, ma1, ma2, ma3)
        msgsb = (mb0, mb1, mb2, mb3)

        c = lax.axis_index("c")
        s = lax.axis_index("s")
        off = s * STRIPE
        ebase = s * EPT

        def striped_parts(stripe_len):
            parts, p = [], 0
            while p < stripe_len:
                plen = min(K, stripe_len - p)
                parts.append((p, plen))
                p += plen
            return parts

        def seed_stripe(acc, stripe_len):
            for p, plen in striped_parts(stripe_len):
                pltpu.sync_copy(bias_hbm.at[pl.ds(off + p, plen)],
                                ma0.at[pl.ds(0, plen)])
                pltpu.sync_copy(ma0.at[pl.ds(0, plen)],
                                acc.at[pl.ds(off + p, plen)])

        for acc in accs:
            @pl.when(s < NS - 1)
            def _():
                seed_stripe(acc, STRIPE)

            @pl.when(s == NS - 1)
            def _():
                seed_stripe(acc, LAST)

        plsc.subcore_barrier()

        def in_descs(i, s2, s4):
            base = ebase + i * K
            sem = sem_in[s4]
            return (
                pltpu.make_async_copy(src_hbm.at[pl.ds(base, K)], srcs[s2], sem),
                pltpu.make_async_copy(vals_hbm.at[pl.ds(base, K)], valss[s2], sem),
                pltpu.make_async_copy(dst_hbm.at[pl.ds(base, K)], dsts[s4], sem),
            )

        def sc_descs(s4):
            return (
                pltpu.make_async_copy(msgsa[s4], accs[0].at[dsts[s4]], sem_sc[s4]),
                pltpu.make_async_copy(msgsb[s4], accs[1].at[dsts[s4]], sem_sc[s4]),
            )

        for d in in_descs(0, 0, 0):
            d.start()
        for d in in_descs(1, 1, 1):
            d.start()
        pltpu.sync_copy(x_hbm.at[pl.ds(c * N, N)], x_v)

        def group_body(g, carry):
            for u in range(4):
                i = 4 * g + u

                @pl.when(i < NCH)
                def _():
                    s2, s4 = u % 2, u
                    for d in in_descs(i, s2, s4):
                        d.wait()

                    src_v, vals_v = srcs[s2], valss[s2]
                    ma_v, mb_v = msgsa[s4], msgsb[s4]

                    @plsc.parallel_loop(0, G, 1, unroll=UNROLL)
                    def _(g2):
                        sl = pl.ds(g2 * _L, _L)
                        packed = plsc.load_gather(x_v, [src_v[sl]])
                        xa = plsc.bitcast(lax.shift_left(packed, 16),
                                          jnp.float32)
                        xb = plsc.bitcast(
                            lax.bitwise_and(packed, jnp.int32(-65536)),
                            jnp.float32)
                        v = vals_v[sl]
                        ma_v[sl] = xa * v
                        mb_v[sl] = xb * v

                    for d in sc_descs(s4):
                        d.start(add=True)

                    n4 = (u + 2) % 4
                    @pl.when(i >= 2)
                    def _():
                        for d in sc_descs(n4):
                            d.wait()

                    @pl.when(i + 2 < NCH)
                    def _():
                        for d in in_descs(i + 2, (u + 2) % 2, n4):
                            d.start()
            return carry

        lax.fori_loop(0, NGRP, group_body, 0)

        for s4 in ((NCH - 2) % 4, (NCH - 1) % 4):
            for d in sc_descs(s4):
                d.wait()

        plsc.subcore_barrier()

        def drain_stripe(acc, bg, stripe_len):
            for p, plen in striped_parts(stripe_len):
                pltpu.sync_copy(acc.at[pl.ds(off + p, plen)],
                                ma0.at[pl.ds(0, plen)])
                pltpu.sync_copy(ma0.at[pl.ds(0, plen)],
                                out_hbm.at[pl.ds(bg * M + off + p, plen)])

        for b_local in range(BPC):
            acc = accs[b_local]
            bg = c * BPC + b_local

            @pl.when(s < NS - 1)
            def _():
                drain_stripe(acc, bg, STRIPE)

            @pl.when(s == NS - 1)
            def _():
                drain_stripe(acc, bg, LAST)

    return k


def kernel(x, values, bias, indices):
    B, N, _ = x.shape
    M = bias.shape[0]
    E = values.shape[0]
    xb = x[:, :, 0].astype(jnp.bfloat16)
    pairs = jnp.stack([xb[0::2], xb[1::2]], axis=-1)
    xp = jax.lax.bitcast_convert_type(pairs, jnp.int32).reshape(-1)
    src = indices[0].astype(jnp.int32)
    dst = indices[1].astype(jnp.int32)
    out = _build(B, N, M, E)(xp, values, src, dst, bias[:, 0])
    return out.reshape(B, M, 1)


# speedup vs baseline: 2.9458x; 1.0016x over previous
import functools

import jax
import jax.numpy as jnp
from jax import lax
from jax.experimental import pallas as pl
from jax.experimental.pallas import tpu as pltpu
from jax.experimental.pallas import tpu_sc as plsc

_L = 16


@functools.lru_cache(maxsize=None)
def _build(B, N, M, E):
    info = plsc.get_sparse_core_info()
    NC, NS = info.num_cores, info.num_subcores
    BPC = B // NC
    assert BPC == 2, "x packing assumes two batch rows per SparseCore"
    EPT = E // NS
    K = 800
    NCH = EPT // K
    assert EPT % K == 0 and K % _L == 0
    NGRP = (NCH + 3) // 4
    G = K // _L
    UNROLL = 10
    assert G % UNROLL == 0

    STRIPE = 6256
    LAST = M - (NS - 1) * STRIPE

    mesh = plsc.VectorSubcoreMesh(core_axis_name="c", subcore_axis_name="s")

    @functools.partial(
        pl.kernel,
        mesh=mesh,
        out_type=jax.ShapeDtypeStruct((B * M,), jnp.float32),
        compiler_params=pltpu.CompilerParams(needs_layout_passes=False),
        scratch_types=(
            [pltpu.VMEM((N,), jnp.int32)]
            + [pltpu.VMEM((K,), jnp.int32) for _ in range(4)]
            + [pltpu.VMEM((K,), jnp.float32) for _ in range(4)]
            + [pltpu.VMEM((K,), jnp.int32) for _ in range(4)]
            + [pltpu.VMEM((K,), jnp.float32) for _ in range(4)]
            + [pltpu.VMEM((K,), jnp.float32) for _ in range(4)]
            + [pltpu.VMEM_SHARED((M,), jnp.float32) for _ in range(BPC)]
            + [pltpu.SemaphoreType.DMA for _ in range(12)]
        ),
    )
    def k(x_hbm, vals_hbm, src_hbm, dst_hbm, bias_hbm, out_hbm,
          x_v, src0, src1, src2, src3, vals0, vals1, vals2, vals3,
          dst0, dst1, dst2, dst3,
          ma0, ma1, ma2, ma3, mb0, mb1, mb2, mb3,
          *accs_and_sems):
        accs = accs_and_sems[:BPC]
        sem_sv = accs_and_sems[BPC:BPC + 4]
        sem_d = accs_and_sems[BPC + 4:BPC + 8]
        sem_sc = accs_and_sems[BPC + 8:BPC + 12]
        srcs = (src0, src1, src2, src3)
        valss = (vals0, vals1, vals2, vals3)
        dsts = (dst0, dst1, dst2, dst3)
        msgsa = (ma0, ma1, ma2, ma3)
        msgsb = (mb0, mb1, mb2, mb3)

        c = lax.axis_index("c")
        s = lax.axis_index("s")
        off = s * STRIPE
        ebase = s * EPT

        def striped_parts(stripe_len):
            parts, p = [], 0
            while p < stripe_len:
                plen = min(K, stripe_len - p)
                parts.append((p, plen))
                p += plen
            return parts

        def seed_stripe(acc, stripe_len):
            for p, plen in striped_parts(stripe_len):
                pltpu.sync_copy(bias_hbm.at[pl.ds(off + p, plen)],
                                ma0.at[pl.ds(0, plen)])
                pltpu.sync_copy(ma0.at[pl.ds(0, plen)],
                                acc.at[pl.ds(off + p, plen)])

        for acc in accs:
            @pl.when(s < NS - 1)
            def _():
                seed_stripe(acc, STRIPE)

            @pl.when(s == NS - 1)
            def _():
                seed_stripe(acc, LAST)

        plsc.subcore_barrier()

        def sv_descs(i, s4):
            base = ebase + i * K
            sem = sem_sv[s4]
            return (
                pltpu.make_async_copy(src_hbm.at[pl.ds(base, K)], srcs[s4], sem),
                pltpu.make_async_copy(vals_hbm.at[pl.ds(base, K)], valss[s4], sem),
            )

        def d_desc(i, s4):
            base = ebase + i * K
            return pltpu.make_async_copy(
                dst_hbm.at[pl.ds(base, K)], dsts[s4], sem_d[s4])

        def sc_descs(s4):
            return (
                pltpu.make_async_copy(msgsa[s4], accs[0].at[dsts[s4]], sem_sc[s4]),
                pltpu.make_async_copy(msgsb[s4], accs[1].at[dsts[s4]], sem_sc[s4]),
            )

        for j in range(4):
            for d in sv_descs(j, j):
                d.start()
        for j in range(2):
            d_desc(j, j).start()
        pltpu.sync_copy(x_hbm.at[pl.ds(c * N, N)], x_v)

        def group_body(g, carry):
            for u in range(4):
                i = 4 * g + u

                @pl.when(i < NCH)
                def _():
                    s4 = u
                    for d in sv_descs(i, s4):
                        d.wait()
                    d_desc(i, s4).wait()

                    src_v, vals_v = srcs[s4], valss[s4]
                    ma_v, mb_v = msgsa[s4], msgsb[s4]

                    @plsc.parallel_loop(0, G, 1, unroll=UNROLL)
                    def _(g2):
                        sl = pl.ds(g2 * _L, _L)
                        packed = plsc.load_gather(x_v, [src_v[sl]])
                        xa = plsc.bitcast(lax.shift_left(packed, 16),
                                          jnp.float32)
                        xb = plsc.bitcast(
                            lax.bitwise_and(packed, jnp.int32(-65536)),
                            jnp.float32)
                        v = vals_v[sl]
                        ma_v[sl] = xa * v
                        mb_v[sl] = xb * v

                    for d in sc_descs(s4):
                        d.start(add=True)

                    n4 = (u + 2) % 4
                    @pl.when(i >= 2)
                    def _():
                        for d in sc_descs(n4):
                            d.wait()

                    @pl.when(i + 2 < NCH)
                    def _():
                        d_desc(i + 2, n4).start()

                    @pl.when(i + 4 < NCH)
                    def _():
                        for d in sv_descs(i + 4, s4):
                            d.start()
            return carry

        lax.fori_loop(0, NGRP, group_body, 0)

        for s4 in ((NCH - 2) % 4, (NCH - 1) % 4):
            for d in sc_descs(s4):
                d.wait()

        plsc.subcore_barrier()

        def drain_stripe(acc, bg, stripe_len):
            for p, plen in striped_parts(stripe_len):
                pltpu.sync_copy(acc.at[pl.ds(off + p, plen)],
                                ma0.at[pl.ds(0, plen)])
                pltpu.sync_copy(ma0.at[pl.ds(0, plen)],
                                out_hbm.at[pl.ds(bg * M + off + p, plen)])

        for b_local in range(BPC):
            acc = accs[b_local]
            bg = c * BPC + b_local

            @pl.when(s < NS - 1)
            def _():
                drain_stripe(acc, bg, STRIPE)

            @pl.when(s == NS - 1)
            def _():
                drain_stripe(acc, bg, LAST)

    return k


def kernel(x, values, bias, indices):
    B, N, _ = x.shape
    M = bias.shape[0]
    E = values.shape[0]
    xb = x[:, :, 0].astype(jnp.bfloat16)
    pairs = jnp.stack([xb[0::2], xb[1::2]], axis=-1)
    xp = jax.lax.bitcast_convert_type(pairs, jnp.int32).reshape(-1)
    src = indices[0].astype(jnp.int32)
    dst = indices[1].astype(jnp.int32)
    out = _build(B, N, M, E)(xp, values, src, dst, bias[:, 0])
    return out.reshape(B, M, 1)
